# baseline (speedup 1.0000x reference)
import jax
import jax.numpy as jnp
from jax import lax
from jax.experimental import pallas as pl
from jax.experimental.pallas import tpu as pltpu
from jax.experimental.pallas import tpu_sc as plsc

DIM = 32
B_TOTAL = 16384 * 50

NC = 2
NS = 16
NW = NC * NS
LANES = 16

A_PER_W = 4
HC = 5
CHUNK = HC * 128
N_CHUNKS = 50 // HC
BLK_W = 128 * 50


def _body(x_hbm, table_hbm, out_hbm, x_v, idx_v, gb0, gb1, tb0, tb1,
          gsem0, gsem1, ssem0, ssem1):
    wid = lax.axis_index("s") * NC + lax.axis_index("c")

    iota = lax.iota(jnp.int32, LANES)
    gbufs = (gb0, gb1)
    tbufs = (tb0, tb1)
    gsems = (gsem0, gsem1)
    ssems = (ssem0, ssem1)

    def gstart(a, cc, b):
        pltpu.make_async_copy(
            table_hbm.at[idx_v.at[pl.ds(cc * CHUNK, CHUNK)]],
            gbufs[b], gsems[b],
        ).start()

    def gwait(b):
        pltpu.make_async_copy(
            table_hbm.at[idx_v.at[pl.ds(0, CHUNK)]], gbufs[b], gsems[b]
        ).wait()

    def sstart(a, cc, b):
        for hh in range(HC):
            h = cc * HC + hh
            for e in range(DIM // 8):
                pltpu.make_async_copy(
                    tbufs[b].at[hh, pl.ds(e * 8, 8), pl.ds(0, 128)],
                    out_hbm.at[h, e, a], ssems[b]
                ).start()

    def swait(b):
        for _ in range(HC * (DIM // 8)):
            pltpu.make_async_copy(
                tbufs[b].at[0, pl.ds(0, 8), pl.ds(0, 128)],
                out_hbm.at[0, 0, 0], ssems[b]
            ).wait()

    def conv(blk):
        @plsc.parallel_loop(0, BLK_W // LANES, unroll=4)
        def body(k):
            h = k // 8
            c0 = (k % 8) * LANES
            cvec = iota + c0
            hvec = jnp.full((LANES,), h, jnp.int32)
            v = plsc.load_gather(x_v, [cvec, hvec]) + 1.0
            v = jnp.where(v != v, 0.0, v)
            idx_v[pl.ds(k * LANES, LANES)] = v.astype(jnp.int32)

    def transpose(b):
        gb, tb = gbufs[b], tbufs[b]
        for hh in range(HC):
            @plsc.parallel_loop(0, 128, unroll=4)
            def tbody(c, hh=hh):
                r = hh * 128 + c
                v0 = gb[r, pl.ds(0, LANES)]
                v1 = gb[r, pl.ds(LANES, LANES)]
                cb = jnp.full((LANES,), c, jnp.int32)
                plsc.store_scatter(tb.at[hh], [iota, cb], v0)
                plsc.store_scatter(tb.at[hh], [iota + LANES, cb], v1)

    def chunk_step(a, cc, b):
        gwait(b)
        swait(b)
        transpose(b)
        sstart(a, cc, b)

    a_last = wid * A_PER_W + (A_PER_W - 1)
    sstart(a_last, N_CHUNKS - 1, 0)
    sstart(a_last, N_CHUNKS - 1, 1)

    def ablock(blk, carry):
        a = wid * A_PER_W + blk
        pltpu.sync_copy(x_hbm.at[pl.ds(a * 128, 128), :], x_v)
        conv(blk)
        gstart(a, 0, 0)
        gstart(a, 1, 1)

        def pair(j, carry2):
            c0 = 2 * j
            chunk_step(a, c0, 0)
            gstart(a, c0 + 2, 0)
            chunk_step(a, c0 + 1, 1)
            gstart(a, c0 + 3, 1)
            return carry2

        lax.fori_loop(0, (N_CHUNKS - 2) // 2, pair, 0)
        chunk_step(a, N_CHUNKS - 2, 0)
        chunk_step(a, N_CHUNKS - 1, 1)
        return carry

    lax.fori_loop(0, A_PER_W, ablock, 0)
    swait(0)
    swait(1)


def kernel(x, table):
    mesh = plsc.VectorSubcoreMesh(core_axis_name="c", subcore_axis_name="s")
    out = pl.kernel(
        _body,
        mesh=mesh,
        out_type=jax.ShapeDtypeStruct((50, DIM // 8, 128, 8, 128),
                                      jnp.float32),
        scratch_types=[
            pltpu.VMEM((128, 50), jnp.float32),
            pltpu.VMEM((BLK_W,), jnp.int32),
            pltpu.VMEM((CHUNK, DIM), jnp.float32),
            pltpu.VMEM((CHUNK, DIM), jnp.float32),
            pltpu.VMEM((HC, DIM, 129), jnp.float32),
            pltpu.VMEM((HC, DIM, 129), jnp.float32),
            pltpu.SemaphoreType.DMA,
            pltpu.SemaphoreType.DMA,
            pltpu.SemaphoreType.DMA,
            pltpu.SemaphoreType.DMA,
        ],
        compiler_params=pltpu.CompilerParams(
            use_tc_tiling_on_sc=False, needs_layout_passes=False
        ),
    )(x, table)
    return jnp.transpose(out, (2, 4, 0, 1, 3)).reshape(16384, 50, DIM)


# speedup vs baseline: 1.0640x; 1.0640x over previous
import jax
import jax.numpy as jnp
from jax import lax
from jax.experimental import pallas as pl
from jax.experimental.pallas import tpu as pltpu
from jax.experimental.pallas import tpu_sc as plsc

DIM = 32
B_TOTAL = 16384 * 50

NC = 2
NS = 16
NW = NC * NS
LANES = 16

A_PER_W = 4
HC = 5
CHUNK = HC * 128
N_CHUNKS = 50 // HC
BLK_W = 128 * 50


def _body(x_hbm, table_hbm, out_hbm, x_v, idx_v, gb0, gb1, tb0, tb1,
          gsem0, gsem1, ssem0, ssem1):
    wid = lax.axis_index("s") * NC + lax.axis_index("c")

    iota = lax.iota(jnp.int32, LANES)
    iota50 = iota * 50
    gbufs = (gb0, gb1)
    tbufs = (tb0, tb1)
    gsems = (gsem0, gsem1)
    ssems = (ssem0, ssem1)

    def gstart(a, cc, b):
        pltpu.make_async_copy(
            table_hbm.at[idx_v.at[pl.ds(cc * CHUNK, CHUNK)]],
            gbufs[b], gsems[b],
        ).start()

    def gwait(b):
        pltpu.make_async_copy(
            table_hbm.at[idx_v.at[pl.ds(0, CHUNK)]], gbufs[b], gsems[b]
        ).wait()

    def sstart(a, cc, b):
        for hh in range(HC):
            h = cc * HC + hh
            for e in range(DIM // 8):
                pltpu.make_async_copy(
                    tbufs[b].at[hh, pl.ds(e * 8, 8), pl.ds(0, 128)],
                    out_hbm.at[h, e, a], ssems[b]
                ).start()

    def swait(b):
        for _ in range(HC * (DIM // 8)):
            pltpu.make_async_copy(
                tbufs[b].at[0, pl.ds(0, 8), pl.ds(0, 128)],
                out_hbm.at[0, 0, 0], ssems[b]
            ).wait()

    def conv(blk):
        @plsc.parallel_loop(0, BLK_W // LANES, unroll=4)
        def body(k):
            h = k // 8
            c0 = (k % 8) * LANES
            src = iota50 + (c0 * 50 + h)
            v = plsc.load_gather(x_v, [src]) + 1.0
            v = jnp.where(v != v, 0.0, v)
            idx_v[pl.ds(k * LANES, LANES)] = v.astype(jnp.int32)

    def transpose(b):
        gb, tb = gbufs[b], tbufs[b]
        for hh in range(HC):
            @plsc.parallel_loop(0, 128, unroll=4)
            def tbody(c, hh=hh):
                r = hh * 128 + c
                v0 = gb[r, pl.ds(0, LANES)]
                v1 = gb[r, pl.ds(LANES, LANES)]
                cb = jnp.full((LANES,), c, jnp.int32)
                plsc.store_scatter(tb.at[hh], [iota, cb], v0)
                plsc.store_scatter(tb.at[hh], [iota + LANES, cb], v1)

    def chunk_step(a, cc, b):
        gwait(b)
        swait(b)
        transpose(b)
        sstart(a, cc, b)

    a_last = wid * A_PER_W + (A_PER_W - 1)
    sstart(a_last, N_CHUNKS - 1, 0)
    sstart(a_last, N_CHUNKS - 1, 1)

    def ablock(blk, carry):
        a = wid * A_PER_W + blk
        pltpu.sync_copy(x_hbm.at[pl.ds(a * BLK_W, BLK_W)], x_v)
        conv(blk)
        gstart(a, 0, 0)
        gstart(a, 1, 1)

        def pair(j, carry2):
            c0 = 2 * j
            chunk_step(a, c0, 0)
            gstart(a, c0 + 2, 0)
            chunk_step(a, c0 + 1, 1)
            gstart(a, c0 + 3, 1)
            return carry2

        lax.fori_loop(0, (N_CHUNKS - 2) // 2, pair, 0)
        chunk_step(a, N_CHUNKS - 2, 0)
        chunk_step(a, N_CHUNKS - 1, 1)
        return carry

    lax.fori_loop(0, A_PER_W, ablock, 0)
    swait(0)
    swait(1)


def kernel(x, table):
    mesh = plsc.VectorSubcoreMesh(core_axis_name="c", subcore_axis_name="s")
    xf = x.reshape(B_TOTAL)
    out = pl.kernel(
        _body,
        mesh=mesh,
        out_type=jax.ShapeDtypeStruct((50, DIM // 8, 128, 8, 128),
                                      jnp.float32),
        scratch_types=[
            pltpu.VMEM((BLK_W,), jnp.float32),
            pltpu.VMEM((BLK_W,), jnp.int32),
            pltpu.VMEM((CHUNK, DIM), jnp.float32),
            pltpu.VMEM((CHUNK, DIM), jnp.float32),
            pltpu.VMEM((HC, DIM, 129), jnp.float32),
            pltpu.VMEM((HC, DIM, 129), jnp.float32),
            pltpu.SemaphoreType.DMA,
            pltpu.SemaphoreType.DMA,
            pltpu.SemaphoreType.DMA,
            pltpu.SemaphoreType.DMA,
        ],
        compiler_params=pltpu.CompilerParams(
            use_tc_tiling_on_sc=False, needs_layout_passes=False
        ),
    )(xf, table)
    return jnp.transpose(out, (2, 4, 0, 1, 3)).reshape(16384, 50, DIM)


# speedup vs baseline: 1.0669x; 1.0027x over previous
import jax
import jax.numpy as jnp
from jax import lax
from jax.experimental import pallas as pl
from jax.experimental.pallas import tpu as pltpu
from jax.experimental.pallas import tpu_sc as plsc

DIM = 32
B_TOTAL = 16384 * 50

NC = 2
NS = 16
NW = NC * NS
LANES = 16

A_PER_W = 4
HC = 5
CHUNK = HC * 128
N_CHUNKS = 50 // HC
BLK_W = 128 * 50


def _body(x_hbm, table_hbm, out_hbm, x_v, idx_v, gb0, gb1, tb0, tb1,
          gsem0, gsem1, ssem0, ssem1):
    wid = lax.axis_index("s") * NC + lax.axis_index("c")

    iota = lax.iota(jnp.int32, LANES)
    iota50 = iota * 50
    gbufs = (gb0, gb1)
    tbufs = (tb0, tb1)
    gsems = (gsem0, gsem1)
    ssems = (ssem0, ssem1)

    def gstart(a, cc, b):
        pltpu.make_async_copy(
            table_hbm.at[idx_v.at[pl.ds(cc * CHUNK, CHUNK)]],
            gbufs[b], gsems[b],
        ).start()

    def gwait(b):
        pltpu.make_async_copy(
            table_hbm.at[idx_v.at[pl.ds(0, CHUNK)]], gbufs[b], gsems[b]
        ).wait()

    def sstart(a, cc, b):
        for hh in range(HC):
            h = cc * HC + hh
            for e in range(DIM // 8):
                pltpu.make_async_copy(
                    tbufs[b].at[hh, pl.ds(e * 8, 8), pl.ds(0, 128)],
                    out_hbm.at[h, e, a], ssems[b]
                ).start()

    def swait(b):
        for _ in range(HC * (DIM // 8)):
            pltpu.make_async_copy(
                tbufs[b].at[0, pl.ds(0, 8), pl.ds(0, 128)],
                out_hbm.at[0, 0, 0], ssems[b]
            ).wait()

    def conv(blk):
        @plsc.parallel_loop(0, BLK_W // LANES, unroll=4)
        def body(k):
            h = k // 8
            c0 = (k % 8) * LANES
            src = iota50 + (c0 * 50 + h)
            v = plsc.load_gather(x_v, [src]) + 1.0
            v = jnp.where(v != v, 0.0, v)
            idx_v[pl.ds(k * LANES, LANES)] = v.astype(jnp.int32)

    def transpose(b):
        gb, tb = gbufs[b], tbufs[b]
        for hh in range(HC):
            @plsc.parallel_loop(0, 128, unroll=8)
            def tbody(c, hh=hh):
                r = hh * 128 + c
                v0 = gb[r, pl.ds(0, LANES)]
                v1 = gb[r, pl.ds(LANES, LANES)]
                cb = jnp.full((LANES,), c, jnp.int32)
                plsc.store_scatter(tb.at[hh], [iota, cb], v0)
                plsc.store_scatter(tb.at[hh], [iota + LANES, cb], v1)

    def chunk_step(a, cc, b):
        gwait(b)
        swait(b)
        transpose(b)
        sstart(a, cc, b)

    a_last = wid * A_PER_W + (A_PER_W - 1)
    sstart(a_last, N_CHUNKS - 1, 0)
    sstart(a_last, N_CHUNKS - 1, 1)

    def ablock(blk, carry):
        a = wid * A_PER_W + blk
        pltpu.sync_copy(x_hbm.at[pl.ds(a * BLK_W, BLK_W)], x_v)
        conv(blk)
        gstart(a, 0, 0)
        gstart(a, 1, 1)

        def pair(j, carry2):
            c0 = 2 * j
            chunk_step(a, c0, 0)
            gstart(a, c0 + 2, 0)
            chunk_step(a, c0 + 1, 1)
            gstart(a, c0 + 3, 1)
            return carry2

        lax.fori_loop(0, (N_CHUNKS - 2) // 2, pair, 0)
        chunk_step(a, N_CHUNKS - 2, 0)
        chunk_step(a, N_CHUNKS - 1, 1)
        return carry

    lax.fori_loop(0, A_PER_W, ablock, 0)
    swait(0)
    swait(1)


def kernel(x, table):
    mesh = plsc.VectorSubcoreMesh(core_axis_name="c", subcore_axis_name="s")
    xf = x.reshape(B_TOTAL)
    out = pl.kernel(
        _body,
        mesh=mesh,
        out_type=jax.ShapeDtypeStruct((50, DIM // 8, 128, 8, 128),
                                      jnp.float32),
        scratch_types=[
            pltpu.VMEM((BLK_W,), jnp.float32),
            pltpu.VMEM((BLK_W,), jnp.int32),
            pltpu.VMEM((CHUNK, DIM), jnp.float32),
            pltpu.VMEM((CHUNK, DIM), jnp.float32),
            pltpu.VMEM((HC, DIM, 129), jnp.float32),
            pltpu.VMEM((HC, DIM, 129), jnp.float32),
            pltpu.SemaphoreType.DMA,
            pltpu.SemaphoreType.DMA,
            pltpu.SemaphoreType.DMA,
            pltpu.SemaphoreType.DMA,
        ],
        compiler_params=pltpu.CompilerParams(
            use_tc_tiling_on_sc=False, needs_layout_passes=False
        ),
    )(xf, table)
    return jnp.transpose(out, (2, 4, 0, 1, 3)).reshape(16384, 50, DIM)
